# Initial kernel scaffold; baseline (speedup 1.0000x reference)
import jax
import jax.numpy as jnp
from jax.experimental import pallas as pl


def kernel(pred, targets):
    raise NotImplementedError("write your pallas kernel here")


# speedup vs baseline: 1.2440x; 1.2440x over previous
import functools

import jax
import jax.numpy as jnp
from jax import lax
from jax.experimental import pallas as pl
from jax.experimental.pallas import tpu as pltpu
from jax.experimental.pallas import tpu_sc as plsc

B = 16
NA = 3
G = 52
C = 80
ROW = C + 5
NCELL = B * NA * G * G
NT = 256
STRIDE = 8.0
LN2 = 0.6931471805599453
AW = (10.0, 16.0, 33.0)
AH = (13.0, 30.0, 23.0)
SAW = tuple(a / STRIDE for a in AW)
SAH = tuple(a / STRIDE for a in AH)

NCH16 = NCELL // 16
NCHUNK = (NCH16 + 31) // 32


def _lnraw(y):
    bits = lax.bitcast_convert_type(y, jnp.int32)
    e = (bits >> 23) - 127
    m = lax.bitcast_convert_type((bits & 0x007FFFFF) | 0x3F800000, jnp.float32)
    big = m > 1.4142135623730951
    m = jnp.where(big, m * 0.5, m)
    e = jnp.where(big, e + 1, e)
    r = (m - 1.0) / (m + 1.0)
    r2 = r * r
    p = r * (2.0 + r2 * (2.0 / 3.0 + r2 * (0.4 + r2 * (2.0 / 7.0 + r2 * (2.0 / 9.0)))))
    return e.astype(jnp.float32) * LN2 + p


def _ln(y):
    return _lnraw(jnp.clip(y, 1e-12, 1.0))


def _sqrt(y):
    return jnp.exp(0.5 * _lnraw(jnp.maximum(y, 1e-36)))


def _chunk_params(tgt_v, off):
    i16 = lax.iota(jnp.int32, 16)
    tb = tgt_v[pl.ds(off, 16)].astype(jnp.int32)
    lab = tgt_v[pl.ds(256 + off, 16)].astype(jnp.int32)
    gx = tgt_v[pl.ds(512 + off, 16)] * float(G)
    gy = tgt_v[pl.ds(768 + off, 16)] * float(G)
    gw = tgt_v[pl.ds(1024 + off, 16)] * float(G)
    gh = tgt_v[pl.ds(1280 + off, 16)] * float(G)
    ious = []
    for a in range(NA):
        inter = jnp.minimum(SAW[a], gw) * jnp.minimum(SAH[a], gh)
        union = SAW[a] * SAH[a] + gw * gh - inter + 1e-16
        ious.append(inter / union)
    zero16 = jnp.zeros((16,), jnp.int32)
    best = zero16
    bv = ious[0]
    best = jnp.where(ious[1] > bv, 1, best)
    bv = jnp.maximum(bv, ious[1])
    best = jnp.where(ious[2] > bv, 2, best)
    gii = jnp.clip(gx.astype(jnp.int32), 0, G - 1)
    gji = jnp.clip(gy.astype(jnp.int32), 0, G - 1)
    cell = ((tb * NA + best) * G + gji) * G + gii
    del i16, zero16
    return dict(tb=tb, lab=lab, gx=gx, gy=gy, gw=gw, gh=gh, ious=ious,
                best=best, gii=gii, gji=gji, cell=cell)


def _body(pred_hbm, tgt_hbm, out_hbm, tgt_v, cells_v, rows_v, bufa_v, bufb_v,
          lbuf_v, part_v, sema, semb, sem3):
    i16 = lax.iota(jnp.int32, 16)
    f16 = i16.astype(jnp.float32)
    ones = jnp.ones((16,), jnp.float32)
    zeros = jnp.zeros((16,), jnp.float32)
    sid = lax.axis_index("s")
    cid = lax.axis_index("c")
    wid = sid * 2 + cid
    m = wid & 15

    pltpu.sync_copy(tgt_hbm, tgt_v)

    def p1(k, carry):
        prm = _chunk_params(tgt_v, k * 16)
        cells_v[pl.ds(k * 16, 16)] = prm["cell"]
        return carry

    lax.fori_loop(0, 16, p1, 0)

    prm = _chunk_params(tgt_v, m * 16)
    cell16 = prm["cell"]
    best = prm["best"]
    gx, gy, gw, gh = prm["gx"], prm["gy"], prm["gw"], prm["gh"]
    flrx = gx.astype(jnp.int32).astype(jnp.float32)
    flry = gy.astype(jnp.int32).astype(jnp.float32)
    tx16 = gx - flrx
    ty16 = (gy - flry + 0.5) * 0.5
    aw16 = jnp.where(best == 0, AW[0], jnp.where(best == 1, AW[1], AW[2]))
    ah16 = jnp.where(best == 0, AH[0], jnp.where(best == 1, AH[1], AH[2]))
    tw16 = _sqrt(gw / (aw16 / STRIDE)) * 0.5
    th16 = _sqrt(gh / (ah16 / STRIDE)) * 0.5
    gif = prm["gii"].astype(jnp.float32)
    gjf = prm["gji"].astype(jnp.float32)

    acc = jnp.zeros((16,), jnp.float32)

    @pl.when(wid < 16)
    def _obj():
        t16 = m * 16 + i16
        def dedup(tp, dup):
            cs = plsc.load_gather(cells_v, [jnp.broadcast_to(tp, (16,))])
            hit = (cell16 == cs) & (tp > t16)
            return jnp.where(hit, 1, dup)

        dup = lax.fori_loop(0, NT, dedup, jnp.zeros((16,), jnp.int32))
        wf = 1.0 - dup.astype(jnp.float32)

        for r in range(5):
            def fire(j, carry):
                k = r * 17 + j
                idx = cell16 * ROW + k
                pltpu.async_copy(pred_hbm.at[idx], rows_v.at[pl.ds(k * 16, 16)], sem3)
                return carry

            lax.fori_loop(0, 17, fire, 0)

            def drain(j, carry):
                k = r * 17 + j
                pltpu.make_async_copy(
                    pred_hbm.at[pl.ds(0, 16)], rows_v.at[pl.ds(k * 16, 16)], sem3
                ).wait()
                return carry

            lax.fori_loop(0, 17, drain, 0)

        idxl = cell16 * ROW + 5 + prm["lab"]
        pltpu.async_copy(pred_hbm.at[idxl], lbuf_v, sem3)
        pltpu.make_async_copy(pred_hbm.at[pl.ds(0, 16)], lbuf_v, sem3).wait()

        px = (rows_v[pl.ds(0, 16)] / STRIDE - gif + 0.5) * 0.5
        py = (rows_v[pl.ds(16, 16)] / STRIDE - gjf + 0.5) * 0.5
        pw = _sqrt(rows_v[pl.ds(32, 16)] / aw16) * 0.5
        ph = _sqrt(rows_v[pl.ds(48, 16)] / ah16) * 0.5
        conf = rows_v[pl.ds(64, 16)]
        lx = wf * (px - tx16) * (px - tx16)
        ly = wf * (py - ty16) * (py - ty16)
        lw = wf * (pw - tw16) * (pw - tw16)
        lh = wf * (ph - th16) * (ph - th16)
        cobj = wf * -_ln(conf)

        def clsbody(k, a):
            v = rows_v[pl.ds(k * 16, 16)]
            return a + -_ln(1.0 - v)

        scls = lax.fori_loop(5, ROW, clsbody, jnp.zeros((16,), jnp.float32))
        pl_ = lbuf_v[...]
        clstot = wf * scls + (-_ln(pl_) + _ln(1.0 - pl_))

        part = zeros
        part = jnp.where(f16 == 1.0, jnp.sum(lx), part)
        part = jnp.where(f16 == 2.0, jnp.sum(ly), part)
        part = jnp.where(f16 == 3.0, jnp.sum(lw), part)
        part = jnp.where(f16 == 4.0, jnp.sum(lh), part)
        part = jnp.where(f16 == 5.0, jnp.sum(cobj), part)
        part = jnp.where(f16 == 6.0, jnp.sum(clstot), part)
        part = jnp.where(f16 == 9.0, jnp.sum(wf), part)
        part_v[...] = part

    @pl.when(wid >= 16)
    def _noobj():
        zcells = []
        for a in range(NA):
            zc = ((prm["tb"] * NA + a) * G + prm["gji"]) * G + prm["gii"]
            zcells.append(zc)
            pltpu.async_copy(
                pred_hbm.at[zc * ROW + 4], rows_v.at[pl.ds(a * 16, 16)], sem3
            )
        zs = jnp.zeros((16,), jnp.float32)
        nz = jnp.zeros((16,), jnp.float32)
        for a in range(NA):
            pltpu.make_async_copy(
                pred_hbm.at[pl.ds(0, 16)], rows_v.at[pl.ds(a * 16, 16)], sem3
            ).wait()
            zb = ((best == a) | (prm["ious"][a] > 0.5)).astype(jnp.float32)
            zs = zs + zb * -_ln(1.0 - rows_v[pl.ds(a * 16, 16)])
            nz = nz + zb
        part = zeros
        part = jnp.where(f16 == 7.0, jnp.sum(zs), part)
        part = jnp.where(f16 == 8.0, jnp.sum(nz), part)
        part_v[...] = part

    gidx = i16 * ROW + 4

    def start(k, buf, sem):
        c = jnp.minimum(wid + 32 * k, NCH16 - 1)
        pltpu.async_copy(pred_hbm.at[pl.ds(c * (16 * ROW), 16 * ROW)], buf, sem)

    def wait(buf, sem):
        pltpu.make_async_copy(pred_hbm.at[pl.ds(0, 16 * ROW)], buf, sem).wait()

    def dense_term(k, buf):
        c = wid + 32 * k
        valid = jnp.broadcast_to(c, (16,)) < NCH16
        w = jnp.where(valid, ones, zeros)
        conf = plsc.load_gather(buf, [gidx])
        return w * -_ln(1.0 - conf)

    start(0, bufa_v, sema)

    def dense_body(i, a):
        wait(bufa_v, sema)
        start(2 * i + 1, bufb_v, semb)
        a = a + dense_term(2 * i, bufa_v)

        @pl.when(i < NCHUNK // 2 - 1)
        def _():
            start(2 * i + 2, bufa_v, sema)

        wait(bufb_v, semb)
        return a + dense_term(2 * i + 1, bufb_v)

    dense = lax.fori_loop(0, NCHUNK // 2, dense_body, jnp.zeros((16,), jnp.float32))
    part_v[...] = part_v[...] + jnp.where(f16 == 0.0, jnp.sum(dense), zeros)
    pltpu.sync_copy(part_v, out_hbm.at[wid])


@functools.partial(jax.jit, donate_argnums=())
def _sc_parts(pred_flat, tgt_flat):
    mesh = plsc.VectorSubcoreMesh(
        core_axis_name="c", subcore_axis_name="s", num_cores=2, num_subcores=16
    )
    return pl.kernel(
        _body,
        out_type=jax.ShapeDtypeStruct((32, 16), jnp.float32),
        mesh=mesh,
        compiler_params=pltpu.CompilerParams(needs_layout_passes=False),
        scratch_types=[
            pltpu.VMEM((6 * NT,), jnp.float32),
            pltpu.VMEM((NT,), jnp.int32),
            pltpu.VMEM((ROW * 16,), jnp.float32),
            pltpu.VMEM((16 * ROW,), jnp.float32),
            pltpu.VMEM((16 * ROW,), jnp.float32),
            pltpu.VMEM((16,), jnp.float32),
            pltpu.VMEM((16,), jnp.float32),
            pltpu.SemaphoreType.DMA,
            pltpu.SemaphoreType.DMA,
            pltpu.SemaphoreType.DMA,
        ],
    )(pred_flat, tgt_flat)


def kernel(pred, targets):
    pred_flat = pred.reshape(-1)
    tgt_flat = jnp.transpose(targets).reshape(-1)
    parts = _sc_parts(pred_flat, tgt_flat)
    s = jnp.sum(parts, axis=0)
    dense, lx, ly, lw, lh, cobj, cls_ = s[0], s[1], s[2], s[3], s[4], s[5], s[6]
    zsum, nzero, nobj = s[7], s[8], s[9]
    total = (
        10.0 * (lx + ly + lw + lh) / nobj
        + cobj / nobj
        + 100.0 * (dense - zsum) / (float(NCELL) - nzero)
        + 10.0 * cls_ / (nobj * float(C))
    )
    return (pred, total.astype(jnp.float32))
